# baseline (speedup 1.0000x reference)
import functools

import jax
import jax.numpy as jnp
from jax.experimental import pallas as pl
from jax.experimental.pallas import tpu as pltpu

_LOG2E = 1.4426950408889634
_CHUNK = 512


def _gat_kernel(h_ref, w_ref, a_ref, adj_hbm, out_ref,
                whb_ref, sq_ref, sk_ref, pb_ref, abuf_ref, sem,
                *, alpha, tq, n_tiles, f_out, n):
    i = pl.program_id(0)

    def _tile_copy(t):
        slot = jax.lax.rem(t, 4)
        return pltpu.make_async_copy(
            adj_hbm.at[pl.ds(t * tq, tq), :], abuf_ref.at[slot], sem.at[slot])

    @pl.when(i == 0)
    def _fetch_first():
        for t in range(min(3, n_tiles)):
            _tile_copy(t).start()

    @pl.when(i == 0)
    def _project():
        a_mat = jnp.concatenate(
            [a_ref[0:f_out, :], a_ref[f_out:2 * f_out, :]], axis=1)
        for c in range(n_tiles):
            hc = h_ref[c * tq:(c + 1) * tq, :]
            wh = jnp.dot(hc, w_ref[...], preferred_element_type=jnp.float32)
            whb_ref[c * tq:(c + 1) * tq, 0:f_out] = wh.astype(jnp.bfloat16)
            sc = jnp.dot(wh, a_mat, preferred_element_type=jnp.float32)
            sq_ref[c * tq:(c + 1) * tq, :] = sc[:, 0:1]
            sk_ref[0:1, c * tq:(c + 1) * tq] = jnp.transpose(sc[:, 1:2])
        lane = jax.lax.broadcasted_iota(jnp.int32, (n, 128), 1)
        whb_ref[:, f_out:f_out + 128] = jnp.where(
            lane == 0, 1.0, 0.0).astype(jnp.bfloat16)

    @pl.when(i > 0)
    def _attend():
        j = i - 1

        @pl.when(j + 3 < n_tiles)
        def _prefetch_next():
            _tile_copy(j + 3).start()

        sk = sk_ref[...]
        sq = sq_ref[pl.ds(j * tq, tq), :]
        rm = sq + jnp.max(sk)
        m = jnp.maximum(rm, alpha * rm)

        _tile_copy(j).wait()
        adj_ref = abuf_ref.at[jax.lax.rem(j, 4)]

        a1 = (sq - m) * _LOG2E
        a2 = (alpha * sq - m) * _LOG2E
        b1 = sk * _LOG2E
        b2 = sk * (alpha * _LOG2E)
        for k in range(n // _CHUNK):
            s0 = k * _CHUNK
            s1 = s0 + _CHUNK
            t = jnp.maximum(a1 + b1[:, s0:s1], a2 + b2[:, s0:s1])
            pb_ref[:, s0:s1] = (jnp.exp2(t)
                                * adj_ref[:, s0:s1]).astype(jnp.bfloat16)

        accf = jnp.dot(pb_ref[...], whb_ref[...],
                       preferred_element_type=jnp.float32)
        denom = accf[:, f_out:f_out + 1]
        out = accf[:, 0:f_out] * pl.reciprocal(denom, approx=False)
        out = jnp.where(out > 0, out, jnp.exp(out) - 1.0)
        out_ref[...] = out


def _row_tile(n, max_tile=512):
    if n <= max_tile:
        return n
    for t in (512, 256, 128):
        if n % t == 0:
            return t
    return n


def kernel(h, W, a, adj):
    alpha = 0.2
    N, f_in = h.shape
    f_out = W.shape[1]

    tq = _row_tile(N)
    n_tiles = N // tq

    cost = pl.CostEstimate(
        flops=2 * N * f_in * f_out + 4 * N * f_out + 2 * N * N * (f_out + 128)
        + 8 * N * N,
        transcendentals=N * N + N * f_out,
        bytes_accessed=4 * (N * N + N * f_in + N * f_out + f_in * f_out),
    )
    body = functools.partial(_gat_kernel, alpha=alpha, tq=tq,
                             n_tiles=n_tiles, f_out=f_out, n=N)
    out = pl.pallas_call(
        body,
        out_shape=jax.ShapeDtypeStruct((N, f_out), jnp.float32),
        grid=(n_tiles + 1,),
        in_specs=[
            pl.BlockSpec((N, f_in), lambda i: (0, 0)),
            pl.BlockSpec((f_in, f_out), lambda i: (0, 0)),
            pl.BlockSpec((2 * f_out, 1), lambda i: (0, 0)),
            pl.BlockSpec(memory_space=pl.ANY),
        ],
        out_specs=pl.BlockSpec(
            (tq, f_out), lambda i: (jnp.maximum(i - 1, 0), 0)),
        scratch_shapes=[
            pltpu.VMEM((N, f_out + 128), jnp.bfloat16),
            pltpu.VMEM((N, 1), jnp.float32),
            pltpu.VMEM((1, N), jnp.float32),
            pltpu.VMEM((tq, N), jnp.bfloat16),
            pltpu.VMEM((4, tq, N), jnp.float32),
            pltpu.SemaphoreType.DMA((4,)),
        ],
        compiler_params=pltpu.CompilerParams(
            dimension_semantics=("arbitrary",)),
        cost_estimate=cost,
    )(h, W, a, adj)
    return out


# speedup vs baseline: 1.0206x; 1.0206x over previous
import functools

import jax
import jax.numpy as jnp
from jax.experimental import pallas as pl
from jax.experimental.pallas import tpu as pltpu

_LOG2E = 1.4426950408889634
_CHUNK = 512


def _gat_kernel(h_ref, w_ref, a_ref, adj_hbm, out_ref,
                whb_ref, sq_ref, sk_ref, pb_ref, abuf_ref, sem,
                *, alpha, tq, n_tiles, f_out, n):
    i = pl.program_id(0)

    def _tile_copy(t):
        slot = jax.lax.rem(t, 3)
        return pltpu.make_async_copy(
            adj_hbm.at[pl.ds(t * tq, tq), :], abuf_ref.at[slot], sem.at[slot])

    @pl.when(i == 0)
    def _fetch_first():
        _tile_copy(0).start()
        if n_tiles > 1:
            _tile_copy(1).start()

    @pl.when(i == 0)
    def _project():
        a_mat = jnp.concatenate(
            [a_ref[0:f_out, :], a_ref[f_out:2 * f_out, :]], axis=1)
        for c in range(n_tiles):
            hc = h_ref[c * tq:(c + 1) * tq, :]
            wh = jnp.dot(hc, w_ref[...], preferred_element_type=jnp.float32)
            whb_ref[c * tq:(c + 1) * tq, 0:f_out] = wh.astype(jnp.bfloat16)
            sc = jnp.dot(wh, a_mat, preferred_element_type=jnp.float32)
            sq_ref[c * tq:(c + 1) * tq, :] = sc[:, 0:1]
            sk_ref[0:1, c * tq:(c + 1) * tq] = jnp.transpose(sc[:, 1:2])
        lane = jax.lax.broadcasted_iota(jnp.int32, (n, 128), 1)
        whb_ref[:, f_out:f_out + 128] = jnp.where(
            lane == 0, 1.0, 0.0).astype(jnp.bfloat16)

    @pl.when(i > 0)
    def _attend():
        j = i - 1

        @pl.when(j + 2 < n_tiles)
        def _prefetch_next():
            _tile_copy(j + 2).start()

        sk = sk_ref[...]
        sq = sq_ref[pl.ds(j * tq, tq), :]
        rm = sq + jnp.max(sk)
        m = jnp.maximum(rm, alpha * rm)

        _tile_copy(j).wait()
        adj_ref = abuf_ref.at[jax.lax.rem(j, 3)]

        a1 = (sq - m) * _LOG2E
        a2 = (alpha * sq - m) * _LOG2E
        b1 = sk * _LOG2E
        b2 = sk * (alpha * _LOG2E)
        for k in range(n // _CHUNK):
            s0 = k * _CHUNK
            s1 = s0 + _CHUNK
            t = jnp.maximum(a1 + b1[:, s0:s1], a2 + b2[:, s0:s1])
            pb_ref[:, s0:s1] = (jnp.exp2(t)
                                * adj_ref[:, s0:s1]).astype(jnp.bfloat16)

        accf = jnp.dot(pb_ref[...], whb_ref[...],
                       preferred_element_type=jnp.float32)
        denom = accf[:, f_out:f_out + 1]
        out = accf[:, 0:f_out] * pl.reciprocal(denom, approx=False)
        out = jnp.where(out > 0, out, jnp.exp(out) - 1.0)
        out_ref[...] = out


def _row_tile(n, max_tile=512):
    if n <= max_tile:
        return n
    for t in (512, 256, 128):
        if n % t == 0:
            return t
    return n


def kernel(h, W, a, adj):
    alpha = 0.2
    N, f_in = h.shape
    f_out = W.shape[1]

    tq = _row_tile(N)
    n_tiles = N // tq

    cost = pl.CostEstimate(
        flops=2 * N * f_in * f_out + 4 * N * f_out + 2 * N * N * (f_out + 128)
        + 8 * N * N,
        transcendentals=N * N + N * f_out,
        bytes_accessed=4 * (N * N + N * f_in + N * f_out + f_in * f_out),
    )
    body = functools.partial(_gat_kernel, alpha=alpha, tq=tq,
                             n_tiles=n_tiles, f_out=f_out, n=N)
    out = pl.pallas_call(
        body,
        out_shape=jax.ShapeDtypeStruct((N, f_out), jnp.float32),
        grid=(n_tiles + 1,),
        in_specs=[
            pl.BlockSpec((N, f_in), lambda i: (0, 0)),
            pl.BlockSpec((f_in, f_out), lambda i: (0, 0)),
            pl.BlockSpec((2 * f_out, 1), lambda i: (0, 0)),
            pl.BlockSpec(memory_space=pl.ANY),
        ],
        out_specs=pl.BlockSpec(
            (tq, f_out), lambda i: (jnp.maximum(i - 1, 0), 0)),
        scratch_shapes=[
            pltpu.VMEM((N, f_out + 128), jnp.bfloat16),
            pltpu.VMEM((N, 1), jnp.float32),
            pltpu.VMEM((1, N), jnp.float32),
            pltpu.VMEM((tq, N), jnp.bfloat16),
            pltpu.VMEM((3, tq, N), jnp.float32),
            pltpu.SemaphoreType.DMA((3,)),
        ],
        compiler_params=pltpu.CompilerParams(
            dimension_semantics=("arbitrary",)),
        cost_estimate=cost,
    )(h, W, a, adj)
    return out
